# baseline (speedup 1.0000x reference)
import jax
import jax.numpy as jnp
from jax import lax
from jax.experimental import pallas as pl
from jax.experimental.pallas import tpu as pltpu

B, N = 128, 8192
CB = 4096
NBLK = N // CB
G = 256
NG = CB // G


def _body(x_ref, m_ref, o_ref, carry_ref):
    c = pl.program_id(0)

    @pl.when(c == 0)
    def _():
        carry_ref[...] = jnp.zeros_like(carry_ref)

    rows = lax.broadcasted_iota(jnp.int32, (G, G), 0)
    cols = lax.broadcasted_iota(jnp.int32, (G, G), 1)
    u = (rows <= cols).astype(jnp.float32)

    masked = x_ref[...] * m_ref[...].astype(jnp.float32)
    off = carry_ref[...]
    for g in range(NG):
        s = jnp.dot(masked[:, g * G:(g + 1) * G], u,
                    preferred_element_type=jnp.float32)
        o_ref[:, g * G:(g + 1) * G] = s + off
        off = off + jnp.broadcast_to(s[:, G - 1:G], (B, G))
    carry_ref[...] = off


def kernel(x, mask):
    return pl.pallas_call(
        _body,
        grid=(NBLK,),
        in_specs=[
            pl.BlockSpec((B, CB), lambda c: (0, c)),
            pl.BlockSpec((B, CB), lambda c: (0, c)),
        ],
        out_specs=pl.BlockSpec((B, CB), lambda c: (0, c)),
        out_shape=jax.ShapeDtypeStruct((B, N), jnp.float32),
        scratch_shapes=[pltpu.VMEM((B, G), jnp.float32)],
    )(x, mask.view(jnp.uint8))
